# Initial kernel scaffold; baseline (speedup 1.0000x reference)
import jax
import jax.numpy as jnp
from jax.experimental import pallas as pl


def kernel(x):
    raise NotImplementedError("write your pallas kernel here")


# speedup vs baseline: 1.1039x; 1.1039x over previous
import numpy as np
import jax
import jax.numpy as jnp
from jax.experimental import pallas as pl
from jax.experimental.pallas import tpu as pltpu

_B = 128
_D = 3 * 224 * 224
_DSUB = _D // 128
_ALPHA = 0.9


def _build_maps():
    perm = np.asarray(
        jax.random.permutation(jax.random.fold_in(jax.random.key(0), 1), _B)
    ).astype(np.int32)
    seen = np.zeros(_B, dtype=bool)
    order, leader, is_start = [], [], []
    for s in range(_B):
        if seen[s]:
            continue
        j = s
        first = True
        while not seen[j]:
            seen[j] = True
            order.append(j)
            leader.append(s)
            is_start.append(1 if first else 0)
            first = False
            j = int(perm[j])
    order = np.asarray(order, np.int32)
    a_idx = np.asarray(leader, np.int32)
    b_idx = perm[order]
    start = np.asarray(is_start, np.int32)
    return order, a_idx, b_idx, start


_ORDER, _A_IDX, _B_IDX, _START = _build_maps()


def _body(a_map, b_map, o_map, start, a_ref, b_ref, o_ref, prev_ref):
    k = pl.program_id(0)

    @pl.when(start[k] == 1)
    def _():
        prev_ref[...] = a_ref[...]

    o_ref[...] = _ALPHA * prev_ref[...] + (1.0 - _ALPHA) * b_ref[...]
    prev_ref[...] = b_ref[...]


def kernel(x):
    x3 = x.reshape(_B, _DSUB, 128)
    grid_spec = pltpu.PrefetchScalarGridSpec(
        num_scalar_prefetch=4,
        grid=(_B,),
        in_specs=[
            pl.BlockSpec((1, _DSUB, 128), lambda k, a, b, o, s: (a[k], 0, 0)),
            pl.BlockSpec((1, _DSUB, 128), lambda k, a, b, o, s: (b[k], 0, 0)),
        ],
        out_specs=pl.BlockSpec((1, _DSUB, 128), lambda k, a, b, o, s: (o[k], 0, 0)),
        scratch_shapes=[pltpu.VMEM((1, _DSUB, 128), jnp.float32)],
    )
    out3 = pl.pallas_call(
        _body,
        grid_spec=grid_spec,
        out_shape=jax.ShapeDtypeStruct((_B, _DSUB, 128), jnp.float32),
    )(
        jnp.asarray(_A_IDX),
        jnp.asarray(_B_IDX),
        jnp.asarray(_ORDER),
        jnp.asarray(_START),
        x3,
        x3,
    )
    return out3.reshape(x.shape)
